# baseline (speedup 1.0000x reference)
import functools

import jax
import jax.numpy as jnp
from jax import lax
from jax.experimental import pallas as pl
from jax.experimental.pallas import tpu as pltpu
from jax.experimental.pallas import tpu_sc as plsc

L = 8
K = 1024
D = 128
LANES = 16

NC = 2
NS = 16
NW = NC * NS

CT = 16
GBUF = 4
OBUF = 2


def _build(n_tokens):
    tpw = n_tokens // NW
    n_chunks = tpw // CT
    mesh = plsc.VectorSubcoreMesh(core_axis_name="c", subcore_axis_name="s")

    @functools.partial(
        pl.kernel,
        mesh=mesh,
        out_type=jax.ShapeDtypeStruct((n_tokens, D), jnp.float32),
        scratch_types=[
            pltpu.VMEM((n_chunks, CT * L), jnp.int32),
            pltpu.VMEM((GBUF, CT * L, D), jnp.float32),
            pltpu.VMEM((OBUF, CT, D), jnp.float32),
            pltpu.SemaphoreType.DMA,
            pltpu.SemaphoreType.DMA,
            pltpu.SemaphoreType.DMA,
            pltpu.SemaphoreType.DMA,
            pltpu.SemaphoreType.DMA,
            pltpu.SemaphoreType.DMA,
        ],
    )
    def lookup(idx_hbm, table_hbm, out_hbm, idx_v, rows_v, out_v,
               g0, g1, g2, g3, o0, o1):
        gsems = (g0, g1, g2, g3)
        osems = (o0, o1)
        wid = lax.axis_index("s") * NC + lax.axis_index("c")
        base = wid * tpw
        lvl_off = (lax.iota(jnp.int32, LANES) & (L - 1)) * K

        pltpu.sync_copy(idx_hbm.at[wid], idx_v)

        @plsc.parallel_loop(0, tpw * L // LANES, 1, unroll=4)
        def _fix(i):
            ci = i // (CT * L // LANES)
            sl = pl.ds((i % (CT * L // LANES)) * LANES, LANES)
            idx_v[ci, sl] = idx_v[ci, sl] + lvl_off

        def gather(ci, b):
            pltpu.async_copy(
                table_hbm.at[idx_v.at[ci]], rows_v.at[b], gsems[b])

        def gather_wait(ci, b):
            pltpu.make_async_copy(
                table_hbm.at[idx_v.at[ci]], rows_v.at[b], gsems[b]).wait()

        def out_start(ci, b):
            pltpu.async_copy(
                out_v.at[b], out_hbm.at[pl.ds(base + ci * CT, CT)], osems[b])

        def out_wait(ci, b):
            pltpu.make_async_copy(
                out_v.at[b], out_hbm.at[pl.ds(base + ci * CT, CT)],
                osems[b]).wait()

        def process(ci, b, ob):
            gather_wait(ci, b)

            @pl.when(ci >= OBUF)
            def _():
                out_wait(ci - OBUF, ob)

            @plsc.parallel_loop(0, CT, 1, unroll=2)
            def _acc(t):
                for d in range(D // LANES):
                    sl = pl.ds(d * LANES, LANES)
                    s = rows_v[b, t * L, sl]
                    for l in range(1, L):
                        s = s + rows_v[b, t * L + l, sl]
                    out_v[ob, t, sl] = s

            out_start(ci, ob)

        gather(0, 0)
        gather(1, 1)

        def chunk_group(g, _):
            ci = g * GBUF
            for b in range(GBUF):

                @pl.when(ci + b + 2 < n_chunks)
                def _():
                    gather(ci + b + 2, (b + 2) % GBUF)

                process(ci + b, b, b % OBUF)
            return 0
        lax.fori_loop(0, n_chunks // GBUF, chunk_group, 0)

        for b in range(OBUF):
            out_wait(n_chunks - OBUF + b, b)

    return lookup


def kernel(x_list, weight):
    b, t, l = x_list.shape
    n = b * t
    n_chunks = n // NW // CT
    idx = x_list.reshape(NW, n_chunks, CT * L)
    table = weight.reshape(l * K, D)
    out = _build(n)(idx, table)
    return out.reshape(b, t, D)
